# baseline (speedup 1.0000x reference)
import jax
import jax.numpy as jnp
import numpy as np
from jax import lax
from jax.experimental import pallas as pl
from jax.experimental.pallas import tpu as pltpu

N_NODES = 10000
N_EDGES = 160000
D_IN = 128
D_EDGE = 16
D_OUT = 4
HIDDEN = 32

_BN = 1000
_BE = 3200
_EPAD = 163840
_BSE = 4096
_CSE = 512
_NT = 1000


def _build_r():
    r = np.zeros((HIDDEN, 128), np.float32)
    for h in range(HIDDEN):
        for o in range(D_OUT):
            r[h, 4 * h + o] = 1.0
    return r


def _build_s():
    s = np.zeros((128, D_OUT), np.float32)
    for h in range(HIDDEN):
        for o in range(D_OUT):
            s[4 * h + o, o] = 1.0
    return s


_R_CONST = _build_r()
_S_CONST = _build_s()


def _k1_body(x_ref, w2p_ref, b2r_ref, wroot_ref, bias_ref,
             tx_ref, xb_ref, out0_ref):
    xv = x_ref[...]
    tx_ref[...] = jnp.dot(xv, w2p_ref[...], preferred_element_type=jnp.float32)
    xb_ref[...] = jnp.dot(xv, b2r_ref[...], preferred_element_type=jnp.float32)
    out0_ref[...] = (
        jnp.dot(xv, wroot_ref[...], preferred_element_type=jnp.float32)
        + bias_ref[...]
    )


def _k1(x, w2p, b2r, wroot, bias2d):
    return pl.pallas_call(
        _k1_body,
        grid=(N_NODES // _BN,),
        in_specs=[
            pl.BlockSpec((_BN, D_IN), lambda i: (i, 0)),
            pl.BlockSpec((D_IN, 128), lambda i: (0, 0)),
            pl.BlockSpec((D_IN, D_OUT), lambda i: (0, 0)),
            pl.BlockSpec((D_IN, D_OUT), lambda i: (0, 0)),
            pl.BlockSpec((1, D_OUT), lambda i: (0, 0)),
        ],
        out_specs=[
            pl.BlockSpec((_BN, 128), lambda i: (i, 0)),
            pl.BlockSpec((_BN, D_OUT), lambda i: (i, 0)),
            pl.BlockSpec((_BN, D_OUT), lambda i: (i, 0)),
        ],
        out_shape=[
            jax.ShapeDtypeStruct((N_NODES, 128), jnp.float32),
            jax.ShapeDtypeStruct((N_NODES, D_OUT), jnp.float32),
            jax.ShapeDtypeStruct((N_NODES, D_OUT), jnp.float32),
        ],
    )(x, w2p, b2r, wroot, bias2d)


def _km_body(tx_ref, xb_ref, src_ref, ea_ref, w1_ref, b1_ref, r_ref, s_ref,
             msgs_ref, g_scr, xbg_scr):
    def body(j, _):
        idx = src_ref[0, 0, j]
        g_scr[pl.ds(j, 1), :] = tx_ref[pl.ds(idx, 1), :]
        xbg_scr[pl.ds(j, 1), :] = xb_ref[pl.ds(idx, 1), :]
        return 0

    lax.fori_loop(0, _BE, body, 0)

    hh = jnp.maximum(
        jnp.dot(ea_ref[...], w1_ref[...], preferred_element_type=jnp.float32)
        + b1_ref[...],
        0.0,
    )
    hrep = jnp.dot(hh, r_ref[...], preferred_element_type=jnp.float32)
    p = g_scr[...] * hrep
    msgs_ref[...] = (
        jnp.dot(p, s_ref[...], preferred_element_type=jnp.float32)
        + xbg_scr[...]
    )


def _km(tx, xb, src3d, edge_attr, w1, b12d, r, s):
    return pl.pallas_call(
        _km_body,
        grid=(N_EDGES // _BE,),
        in_specs=[
            pl.BlockSpec((N_NODES, 128), lambda i: (0, 0)),
            pl.BlockSpec((N_NODES, D_OUT), lambda i: (0, 0)),
            pl.BlockSpec((1, 1, _BE), lambda i: (i, 0, 0),
                         memory_space=pltpu.SMEM),
            pl.BlockSpec((_BE, D_EDGE), lambda i: (i, 0)),
            pl.BlockSpec((D_EDGE, HIDDEN), lambda i: (0, 0)),
            pl.BlockSpec((1, HIDDEN), lambda i: (0, 0)),
            pl.BlockSpec((HIDDEN, 128), lambda i: (0, 0)),
            pl.BlockSpec((128, D_OUT), lambda i: (0, 0)),
        ],
        out_specs=pl.BlockSpec((_BE, D_OUT), lambda i: (i, 0)),
        out_shape=jax.ShapeDtypeStruct((N_EDGES, D_OUT), jnp.float32),
        scratch_shapes=[
            pltpu.VMEM((_BE, 128), jnp.float32),
            pltpu.VMEM((_BE, D_OUT), jnp.float32),
        ],
    )(tx, xb, src3d, edge_attr, w1, b12d, r, s)


def _ks_body(out0_ref, msgs_ref, dst_ref, out_ref):
    nt = pl.program_id(0)
    eb = pl.program_id(1)

    @pl.when(eb == 0)
    def _():
        out_ref[...] = out0_ref[...]

    base = nt * _NT
    acc = jnp.zeros((_NT, D_OUT), jnp.float32)
    for c in range(_BSE // _CSE):
        dv = dst_ref[0, 0, pl.ds(c * _CSE, _CSE)]
        node_ids = base + jax.lax.broadcasted_iota(jnp.int32, (_CSE, _NT), 1)
        onehot = (dv[:, None] == node_ids).astype(jnp.float32)
        mchunk = msgs_ref[pl.ds(c * _CSE, _CSE), :]
        acc = acc + jax.lax.dot_general(
            onehot, mchunk,
            dimension_numbers=(((0,), (0,)), ((), ())),
            preferred_element_type=jnp.float32,
        )
    out_ref[...] += acc


def _ks(out0, msgs_pad, dst3d):
    return pl.pallas_call(
        _ks_body,
        grid=(N_NODES // _NT, _EPAD // _BSE),
        in_specs=[
            pl.BlockSpec((_NT, D_OUT), lambda nt, eb: (nt, 0)),
            pl.BlockSpec((_BSE, D_OUT), lambda nt, eb: (eb, 0)),
            pl.BlockSpec((1, 1, _BSE), lambda nt, eb: (eb, 0, 0)),
        ],
        out_specs=pl.BlockSpec((_NT, D_OUT), lambda nt, eb: (nt, 0)),
        out_shape=jax.ShapeDtypeStruct((N_NODES, D_OUT), jnp.float32),
        compiler_params=pltpu.CompilerParams(
            dimension_semantics=("parallel", "arbitrary"),
        ),
    )(out0, msgs_pad, dst3d)


def kernel(x, edge_attr, W1, b1, W2, b2, W_root, bias, edge_index):
    w2p = W2.reshape(HIDDEN, D_IN, D_OUT).transpose(1, 0, 2).reshape(D_IN, 128)
    b2r = b2.reshape(D_IN, D_OUT)
    bias2d = bias.reshape(1, D_OUT)
    b12d = b1.reshape(1, HIDDEN)
    src3d = edge_index[0].reshape(N_EDGES // _BE, 1, _BE)
    dst_pad = jnp.concatenate(
        [edge_index[1],
         jnp.full((_EPAD - N_EDGES,), N_NODES, jnp.int32)])
    dst3d = dst_pad.reshape(_EPAD // _BSE, 1, _BSE)
    r = jnp.asarray(_R_CONST)
    s = jnp.asarray(_S_CONST)

    tx, xb, out0 = _k1(x, w2p, b2r, W_root, bias2d)
    msgs = _km(tx, xb, src3d, edge_attr, W1, b12d, r, s)
    msgs_pad = jnp.concatenate(
        [msgs, jnp.zeros((_EPAD - N_EDGES, D_OUT), jnp.float32)])
    return _ks(out0, msgs_pad, dst3d)
